# baseline (speedup 1.0000x reference)
import functools

import jax
import jax.numpy as jnp
from jax import lax
from jax.experimental import pallas as pl
from jax.experimental.pallas import tpu as pltpu
from jax.experimental.pallas import tpu_sc as plsc

_F32 = jnp.float32
_I32 = jnp.int32

_N = 10000
_D = 128
_NP = 10112
_RPS = _NP // 16
_BLK = _NP // 8
_CH = 128
_NW = 32


def _dot(a, b, prec):
    return lax.dot_general(a, b, (((1,), (0,)), ((), ())), precision=prec,
                           preferred_element_type=_F32)


def _k1_body(x_ref, w_ref, asw_ref, adw_ref, rep_ref, t1_ref, ad1_ref):
    h = _dot(x_ref[...], w_ref[...], lax.Precision.DEFAULT)
    as1 = _dot(h, asw_ref[...], lax.Precision.HIGHEST)
    ad1 = _dot(h, adw_ref[...], lax.Precision.HIGHEST)
    as1r = _dot(as1, rep_ref[...], lax.Precision.HIGHEST)
    ad1r = _dot(ad1, rep_ref[...], lax.Precision.HIGHEST)
    t1_ref[...] = jnp.concatenate([h, as1r], axis=1)
    ad1_ref[...] = ad1r


def _k1(x_pad, W1, asw, adw, rep):
    return pl.pallas_call(
        _k1_body,
        grid=(8,),
        in_specs=[pl.BlockSpec((_BLK, _D), lambda i: (i, 0)),
                  pl.BlockSpec((_D, 64), lambda i: (0, 0)),
                  pl.BlockSpec((64, 8), lambda i: (0, 0)),
                  pl.BlockSpec((64, 8), lambda i: (0, 0)),
                  pl.BlockSpec((8, 64), lambda i: (0, 0))],
        out_specs=[pl.BlockSpec((_BLK, 128), lambda i: (i, 0)),
                   pl.BlockSpec((_BLK, 64), lambda i: (i, 0))],
        out_shape=[jax.ShapeDtypeStruct((_NP, 128), _F32),
                   jax.ShapeDtypeStruct((_NP, 64), _F32)],
    )(x_pad, W1, asw, adw, rep)


def _sc_mesh():
    return plsc.VectorSubcoreMesh(core_axis_name="c", subcore_axis_name="s")


def _make_k2(cpt):
    @functools.partial(
        pl.kernel,
        mesh=_sc_mesh(),
        compiler_params=pltpu.CompilerParams(use_tc_tiling_on_sc=False),
        out_type=jax.ShapeDtypeStruct((2, _NP, 128), _F32),
        scratch_types=[
            pltpu.VMEM((_CH,), _I32),
            pltpu.VMEM((_CH,), _I32),
            pltpu.VMEM((_CH, 128), _F32),
            pltpu.VMEM((_CH, 64), _F32),
            pltpu.VMEM((_CH, 128), _F32),
            pltpu.VMEM_SHARED((_NP, 128), _F32),
            pltpu.SemaphoreType.DMA,
            pltpu.SemaphoreType.DMA,
        ],
    )
    def k2(t1, ad1, srcx, dstx, z128, acco, sidx_v, didx_v, rows_v, adr_v,
           msg_v, acc_sh, sem1, sem2):
        c = lax.axis_index("c")
        s = lax.axis_index("s")
        wid = s * 2 + c
        ro = pl.multiple_of(s * _RPS, 8)
        pltpu.sync_copy(z128.at[pl.ds(ro, _RPS)], acc_sh.at[pl.ds(ro, _RPS)])
        plsc.subcore_barrier()

        def chunk(g, carry):
            base = pl.multiple_of((wid * cpt + g) * _CH, _CH)
            pltpu.sync_copy(srcx.at[pl.ds(base, _CH)], sidx_v)
            pltpu.sync_copy(dstx.at[pl.ds(base, _CH)], didx_v)
            cp1 = pltpu.async_copy(t1.at[sidx_v], rows_v, sem1)
            cp2 = pltpu.async_copy(ad1.at[didx_v], adr_v, sem2)
            cp1.wait()
            cp2.wait()
            for i in range(_CH):
                for q in range(4):
                    lo = 16 * q
                    aq = rows_v[i, 64 + lo:80 + lo] + adr_v[i, lo:lo + 16]
                    aq = jnp.maximum(aq, 0.2 * aq)
                    mq = jnp.exp(aq)
                    msg_v[i, lo:lo + 16] = rows_v[i, lo:lo + 16] * mq
                    msg_v[i, 64 + lo:80 + lo] = mq
            pltpu.sync_copy(msg_v, acc_sh.at[didx_v], add=True)
            return carry

        lax.fori_loop(0, cpt, chunk, 0)
        plsc.subcore_barrier()
        pltpu.sync_copy(acc_sh.at[pl.ds(ro, _RPS)], acco.at[c, pl.ds(ro, _RPS)])

    return k2


def _k3_body(acc_ref, b1_ref, w2_ref, asw_ref, adw_ref, rep_ref, sel_ref,
             t2_ref, ad2_ref):
    a = acc_ref[0] + acc_ref[1]
    den = jnp.maximum(_dot(a[:, 64:128], sel_ref[...],
                           lax.Precision.HIGHEST), 1e-30)
    denr = _dot(den, rep_ref[...], lax.Precision.HIGHEST)
    g = a[:, 0:64] / denr + b1_ref[...]
    g = jnp.where(g > 0, g, jnp.exp(g) - 1.0)
    h2 = _dot(g, w2_ref[...], lax.Precision.DEFAULT)
    as2 = _dot(h2, asw_ref[...], lax.Precision.HIGHEST)
    ad2 = _dot(h2, adw_ref[...], lax.Precision.HIGHEST)
    t2_ref[...] = jnp.concatenate(
        [h2, jnp.broadcast_to(as2, (as2.shape[0], 16))], axis=1)
    ad2_ref[...] = jnp.broadcast_to(ad2, (ad2.shape[0], 16))


def _k3(acco1, b1, W2, asw2, adw2, rep, sel):
    return pl.pallas_call(
        _k3_body,
        grid=(8,),
        in_specs=[pl.BlockSpec((2, _BLK, 128), lambda i: (0, i, 0)),
                  pl.BlockSpec((1, 64), lambda i: (0, 0)),
                  pl.BlockSpec((64, 32), lambda i: (0, 0)),
                  pl.BlockSpec((32, 1), lambda i: (0, 0)),
                  pl.BlockSpec((32, 1), lambda i: (0, 0)),
                  pl.BlockSpec((8, 64), lambda i: (0, 0)),
                  pl.BlockSpec((64, 8), lambda i: (0, 0))],
        out_specs=[pl.BlockSpec((_BLK, 48), lambda i: (i, 0)),
                   pl.BlockSpec((_BLK, 16), lambda i: (i, 0))],
        out_shape=[jax.ShapeDtypeStruct((_NP, 48), _F32),
                   jax.ShapeDtypeStruct((_NP, 16), _F32)],
    )(acco1, b1, W2, asw2, adw2, rep, sel)


def _make_k4(cpt):
    @functools.partial(
        pl.kernel,
        mesh=_sc_mesh(),
        compiler_params=pltpu.CompilerParams(use_tc_tiling_on_sc=False),
        out_type=jax.ShapeDtypeStruct((2, _NP, 48), _F32),
        scratch_types=[
            pltpu.VMEM((_CH,), _I32), pltpu.VMEM((_CH,), _I32),
            pltpu.VMEM((_CH,), _I32), pltpu.VMEM((_CH,), _I32),
            pltpu.VMEM((_CH, 48), _F32), pltpu.VMEM((_CH, 48), _F32),
            pltpu.VMEM((_CH, 16), _F32), pltpu.VMEM((_CH, 16), _F32),
            pltpu.VMEM((_CH, 48), _F32), pltpu.VMEM((_CH, 48), _F32),
            pltpu.VMEM_SHARED((_NP, 48), _F32),
            pltpu.SemaphoreType.DMA, pltpu.SemaphoreType.DMA,
            pltpu.SemaphoreType.DMA, pltpu.SemaphoreType.DMA,
        ],
    )
    def k4(t2, ad2, srcx, dstx, z48, acco, sidx_a, sidx_b, didx_a, didx_b,
           rows_a, rows_b, adr_a, adr_b, msg_a, msg_b, acc_sh,
           sem1a, sem1b, sem2a, sem2b):
        sidx_v = (sidx_a, sidx_b)
        didx_v = (didx_a, didx_b)
        rows_v = (rows_a, rows_b)
        adr_v = (adr_a, adr_b)
        msg_v = (msg_a, msg_b)
        sem1 = (sem1a, sem1b)
        sem2 = (sem2a, sem2b)
        c = lax.axis_index("c")
        s = lax.axis_index("s")
        wid = s * 2 + c
        ro = pl.multiple_of(s * _RPS, 8)
        pltpu.sync_copy(z48.at[pl.ds(ro, _RPS)], acc_sh.at[pl.ds(ro, _RPS)])
        plsc.subcore_barrier()

        def start(b, g):
            base = pl.multiple_of((wid * cpt + g) * _CH, _CH)
            pltpu.sync_copy(srcx.at[pl.ds(base, _CH)], sidx_v[b])
            pltpu.sync_copy(dstx.at[pl.ds(base, _CH)], didx_v[b])
            cp1 = pltpu.async_copy(t2.at[sidx_v[b]], rows_v[b], sem1[b])
            cp2 = pltpu.async_copy(ad2.at[didx_v[b]], adr_v[b], sem2[b])
            return cp1, cp2

        def finish(b, cps):
            cps[0].wait()
            cps[1].wait()
            rv, mv = rows_v[b], msg_v[b]
            for i in range(_CH):
                a16 = rv[i, 32:48] + adr_v[b][i, 0:16]
                a16 = jnp.maximum(a16, 0.2 * a16)
                m = jnp.exp(a16)
                mv[i, 0:16] = rv[i, 0:16] * m
                mv[i, 16:32] = rv[i, 16:32] * m
                mv[i, 32:48] = m
            pltpu.sync_copy(mv, acc_sh.at[didx_v[b]], add=True)

        def pair(k, carry):
            cpa = start(0, 2 * k)
            cpb = start(1, 2 * k + 1)
            finish(0, cpa)
            finish(1, cpb)
            return carry

        lax.fori_loop(0, cpt // 2, pair, 0)
        plsc.subcore_barrier()
        pltpu.sync_copy(acc_sh.at[pl.ds(ro, _RPS)], acco.at[c, pl.ds(ro, _RPS)])

    return k4


def _k5_body(acc_ref, b2_ref, o_ref):
    i = pl.program_id(0)
    a = acc_ref[0] + acc_ref[1]
    den = jnp.maximum(a[:, 32:33], 1e-30)
    h2 = a[:, 0:32] / den + b2_ref[...]
    rowid = i * _BLK + lax.broadcasted_iota(_I32, (_BLK, 1), 0)
    h2 = jnp.where(rowid < _N, h2, 0.0)
    part = jnp.sum(h2.reshape(4, _BLK // 4, 32), axis=0)
    blocksum = jnp.sum(part, axis=0, keepdims=True)

    @pl.when(i == 0)
    def _():
        o_ref[...] = blocksum

    @pl.when(i > 0)
    def _():
        o_ref[...] += blocksum


def _k5(acco2, b2):
    return pl.pallas_call(
        _k5_body,
        grid=(8,),
        in_specs=[pl.BlockSpec((2, _BLK, 48), lambda i: (0, i, 0)),
                  pl.BlockSpec((1, 32), lambda i: (0, 0))],
        out_specs=pl.BlockSpec((1, 32), lambda i: (0, 0)),
        out_shape=jax.ShapeDtypeStruct((1, 32), _F32),
    )(acco2, b2)


def _head_kernel(pooled_ref, wr_ref, br_ref, o_ref):
    o_ref[...] = (jnp.sum(pooled_ref[...] * wr_ref[...][:, 0][None, :], axis=1,
                          keepdims=True) + br_ref[...][None, :])


def kernel(x, edge_index, W1, att_src1, att_dst1, b1, W2, att_src2, att_dst2,
           b2, Wr, br):
    n = x.shape[0]
    e = edge_index.shape[1]
    etot = e + n
    quant = _NW * _CH * 2
    ep = -(-etot // quant) * quant
    cpt = ep // (_NW * _CH)

    loops = jnp.arange(n, dtype=edge_index.dtype)
    padi = jnp.full((ep - etot,), n, dtype=edge_index.dtype)
    srcx = jnp.concatenate([edge_index[0], loops, padi])
    dstx = jnp.concatenate([edge_index[1], loops, padi])

    x_pad = jnp.pad(x, ((0, _NP - n), (0, 0)))
    ar = jnp.arange(64)
    ar8 = jnp.arange(8)
    asw1 = jnp.zeros((64, 8), _F32).at[ar, ar // 8].set(att_src1.reshape(64))
    adw1 = jnp.zeros((64, 8), _F32).at[ar, ar // 8].set(att_dst1.reshape(64))
    rep = jnp.zeros((8, 64), _F32).at[ar // 8, ar].set(1.0)
    sel = jnp.zeros((64, 8), _F32).at[8 * ar8, ar8].set(1.0)
    z128 = jnp.zeros((_NP, 128), _F32)
    z48 = jnp.zeros((_NP, 48), _F32)

    t1, ad1r = _k1(x_pad, W1, asw1, adw1, rep)
    acco1 = _make_k2(cpt)(t1, ad1r, srcx, dstx, z128)
    t2, ad2r = _k3(acco1, b1.reshape(1, 64), W2, att_src2.reshape(32, 1),
                   att_dst2.reshape(32, 1), rep, sel)
    acco2 = _make_k4(cpt)(t2, ad2r, srcx, dstx, z48)
    pooled = _k5(acco2, b2.reshape(1, 32))
    out = pl.pallas_call(
        _head_kernel,
        out_shape=jax.ShapeDtypeStruct((1, 1), _F32),
    )(pooled, Wr, br)
    return out


# speedup vs baseline: 1.4388x; 1.4388x over previous
import functools

import jax
import jax.numpy as jnp
from jax import lax
from jax.experimental import pallas as pl
from jax.experimental.pallas import tpu as pltpu
from jax.experimental.pallas import tpu_sc as plsc

_F32 = jnp.float32
_I32 = jnp.int32

_N = 10000
_D = 128
_NP = 10112
_RPS = _NP // 16
_BLK = _NP // 8
_CH = 128
_NW = 32


def _dot(a, b, prec):
    return lax.dot_general(a, b, (((1,), (0,)), ((), ())), precision=prec,
                           preferred_element_type=_F32)


def _k1_body(x_ref, w_ref, asw_ref, adw_ref, rep_ref, t1_ref, ad1_ref):
    h = _dot(x_ref[...], w_ref[...], lax.Precision.DEFAULT)
    as1 = _dot(h, asw_ref[...], lax.Precision.HIGHEST)
    ad1 = _dot(h, adw_ref[...], lax.Precision.HIGHEST)
    as1r = _dot(as1, rep_ref[...], lax.Precision.HIGHEST)
    ad1r = _dot(ad1, rep_ref[...], lax.Precision.HIGHEST)
    t1_ref[...] = jnp.concatenate([h, as1r], axis=1)
    ad1_ref[...] = ad1r


def _k1(x_pad, W1, asw, adw, rep):
    return pl.pallas_call(
        _k1_body,
        grid=(8,),
        in_specs=[pl.BlockSpec((_BLK, _D), lambda i: (i, 0)),
                  pl.BlockSpec((_D, 64), lambda i: (0, 0)),
                  pl.BlockSpec((64, 8), lambda i: (0, 0)),
                  pl.BlockSpec((64, 8), lambda i: (0, 0)),
                  pl.BlockSpec((8, 64), lambda i: (0, 0))],
        out_specs=[pl.BlockSpec((_BLK, 128), lambda i: (i, 0)),
                   pl.BlockSpec((_BLK, 64), lambda i: (i, 0))],
        out_shape=[jax.ShapeDtypeStruct((_NP, 128), _F32),
                   jax.ShapeDtypeStruct((_NP, 64), _F32)],
    )(x_pad, W1, asw, adw, rep)


def _sc_mesh():
    return plsc.VectorSubcoreMesh(core_axis_name="c", subcore_axis_name="s")


def _make_k2(cpt):
    @functools.partial(
        pl.kernel,
        mesh=_sc_mesh(),
        compiler_params=pltpu.CompilerParams(use_tc_tiling_on_sc=False),
        out_type=jax.ShapeDtypeStruct((2, _NP, 128), _F32),
        scratch_types=[
            pltpu.VMEM((8, _CH), _I32),
            pltpu.VMEM((8, _CH), _I32),
            pltpu.VMEM((_CH, 128), _F32),
            pltpu.VMEM((_CH, 64), _F32),
            pltpu.VMEM((_CH, 128), _F32),
            pltpu.VMEM_SHARED((_NP, 128), _F32),
            pltpu.SemaphoreType.DMA,
            pltpu.SemaphoreType.DMA,
        ],
    )
    def k2(t1, ad1, srcx2, dstx2, z128, acco, sidx_sl, didx_sl, rows_v, adr_v,
           msg_v, acc_sh, sem1, sem2):
        c = lax.axis_index("c")
        s = lax.axis_index("s")
        wid = s * 2 + c
        ro = pl.multiple_of(s * _RPS, 8)
        pltpu.sync_copy(z128.at[pl.ds(ro, _RPS)], acc_sh.at[pl.ds(ro, _RPS)])
        plsc.subcore_barrier()

        def do_chunk(k):
            cp1 = pltpu.async_copy(t1.at[sidx_sl.at[k]], rows_v, sem1)
            cp2 = pltpu.async_copy(ad1.at[didx_sl.at[k]], adr_v, sem2)
            cp1.wait()
            cp2.wait()
            for i in range(_CH):
                for q in range(4):
                    lo = 16 * q
                    aq = rows_v[i, 64 + lo:80 + lo] + adr_v[i, lo:lo + 16]
                    aq = jnp.maximum(aq, 0.2 * aq)
                    mq = jnp.exp(aq)
                    msg_v[i, lo:lo + 16] = rows_v[i, lo:lo + 16] * mq
                    msg_v[i, 64 + lo:80 + lo] = mq
            pltpu.sync_copy(msg_v, acc_sh.at[didx_sl.at[k]], add=True)

        def load_slab(row, nrows):
            pltpu.sync_copy(srcx2.at[pl.ds(row, nrows)], sidx_sl.at[pl.ds(0, nrows)])
            pltpu.sync_copy(dstx2.at[pl.ds(row, nrows)], didx_sl.at[pl.ds(0, nrows)])

        def chunk_body(k, carry):
            do_chunk(k)
            return carry

        def slab(t, carry):
            load_slab(wid * cpt + t * 8, 8)
            lax.fori_loop(0, 8, chunk_body, 0)
            return carry

        lax.fori_loop(0, cpt // 8, slab, 0)
        tail = cpt % 8
        if tail:
            load_slab(wid * cpt + (cpt - tail), tail)
            lax.fori_loop(0, tail, chunk_body, 0)
        plsc.subcore_barrier()
        pltpu.sync_copy(acc_sh.at[pl.ds(ro, _RPS)], acco.at[c, pl.ds(ro, _RPS)])

    return k2


def _k3_body(acc_ref, b1_ref, w2_ref, asw_ref, adw_ref, rep_ref, sel_ref,
             t2_ref, ad2_ref):
    a = acc_ref[0] + acc_ref[1]
    den = jnp.maximum(_dot(a[:, 64:128], sel_ref[...],
                           lax.Precision.HIGHEST), 1e-30)
    denr = _dot(den, rep_ref[...], lax.Precision.HIGHEST)
    g = a[:, 0:64] / denr + b1_ref[...]
    g = jnp.where(g > 0, g, jnp.exp(g) - 1.0)
    h2 = _dot(g, w2_ref[...], lax.Precision.DEFAULT)
    as2 = _dot(h2, asw_ref[...], lax.Precision.HIGHEST)
    ad2 = _dot(h2, adw_ref[...], lax.Precision.HIGHEST)
    t2_ref[...] = jnp.concatenate(
        [h2, jnp.broadcast_to(as2, (as2.shape[0], 16))], axis=1)
    ad2_ref[...] = jnp.broadcast_to(ad2, (ad2.shape[0], 16))


def _k3(acco1, b1, W2, asw2, adw2, rep, sel):
    return pl.pallas_call(
        _k3_body,
        grid=(8,),
        in_specs=[pl.BlockSpec((2, _BLK, 128), lambda i: (0, i, 0)),
                  pl.BlockSpec((1, 64), lambda i: (0, 0)),
                  pl.BlockSpec((64, 32), lambda i: (0, 0)),
                  pl.BlockSpec((32, 1), lambda i: (0, 0)),
                  pl.BlockSpec((32, 1), lambda i: (0, 0)),
                  pl.BlockSpec((8, 64), lambda i: (0, 0)),
                  pl.BlockSpec((64, 8), lambda i: (0, 0))],
        out_specs=[pl.BlockSpec((_BLK, 48), lambda i: (i, 0)),
                   pl.BlockSpec((_BLK, 16), lambda i: (i, 0))],
        out_shape=[jax.ShapeDtypeStruct((_NP, 48), _F32),
                   jax.ShapeDtypeStruct((_NP, 16), _F32)],
    )(acco1, b1, W2, asw2, adw2, rep, sel)


def _make_k4(cpt):
    @functools.partial(
        pl.kernel,
        mesh=_sc_mesh(),
        compiler_params=pltpu.CompilerParams(use_tc_tiling_on_sc=False),
        out_type=jax.ShapeDtypeStruct((2, _NP, 48), _F32),
        scratch_types=[
            pltpu.VMEM((8, _CH), _I32),
            pltpu.VMEM((8, _CH), _I32),
            pltpu.VMEM((_CH, 48), _F32),
            pltpu.VMEM((_CH, 16), _F32),
            pltpu.VMEM((_CH, 48), _F32),
            pltpu.VMEM_SHARED((_NP, 48), _F32),
            pltpu.SemaphoreType.DMA,
            pltpu.SemaphoreType.DMA,
        ],
    )
    def k4(t2, ad2, srcx2, dstx2, z48, acco, sidx_sl, didx_sl, rows_v, adr_v,
           msg_v, acc_sh, sem1, sem2):
        c = lax.axis_index("c")
        s = lax.axis_index("s")
        wid = s * 2 + c
        ro = pl.multiple_of(s * _RPS, 8)
        pltpu.sync_copy(z48.at[pl.ds(ro, _RPS)], acc_sh.at[pl.ds(ro, _RPS)])
        plsc.subcore_barrier()

        def do_chunk(k):
            cp1 = pltpu.async_copy(t2.at[sidx_sl.at[k]], rows_v, sem1)
            cp2 = pltpu.async_copy(ad2.at[didx_sl.at[k]], adr_v, sem2)
            cp1.wait()
            cp2.wait()
            for i in range(_CH):
                a16 = rows_v[i, 32:48] + adr_v[i, 0:16]
                a16 = jnp.maximum(a16, 0.2 * a16)
                m = jnp.exp(a16)
                msg_v[i, 0:16] = rows_v[i, 0:16] * m
                msg_v[i, 16:32] = rows_v[i, 16:32] * m
                msg_v[i, 32:48] = m
            pltpu.sync_copy(msg_v, acc_sh.at[didx_sl.at[k]], add=True)

        def load_slab(row, nrows):
            pltpu.sync_copy(srcx2.at[pl.ds(row, nrows)], sidx_sl.at[pl.ds(0, nrows)])
            pltpu.sync_copy(dstx2.at[pl.ds(row, nrows)], didx_sl.at[pl.ds(0, nrows)])

        def chunk_body(k, carry):
            do_chunk(k)
            return carry

        def slab(t, carry):
            load_slab(wid * cpt + t * 8, 8)
            lax.fori_loop(0, 8, chunk_body, 0)
            return carry

        lax.fori_loop(0, cpt // 8, slab, 0)
        tail = cpt % 8
        if tail:
            load_slab(wid * cpt + (cpt - tail), tail)
            lax.fori_loop(0, tail, chunk_body, 0)
        plsc.subcore_barrier()
        pltpu.sync_copy(acc_sh.at[pl.ds(ro, _RPS)], acco.at[c, pl.ds(ro, _RPS)])

    return k4


def _k5_body(acc_ref, b2_ref, o_ref):
    i = pl.program_id(0)
    a = acc_ref[0] + acc_ref[1]
    den = jnp.maximum(a[:, 32:33], 1e-30)
    h2 = a[:, 0:32] / den + b2_ref[...]
    rowid = i * _BLK + lax.broadcasted_iota(_I32, (_BLK, 1), 0)
    h2 = jnp.where(rowid < _N, h2, 0.0)
    part = jnp.sum(h2.reshape(4, _BLK // 4, 32), axis=0)
    blocksum = jnp.sum(part, axis=0, keepdims=True)

    @pl.when(i == 0)
    def _():
        o_ref[...] = blocksum

    @pl.when(i > 0)
    def _():
        o_ref[...] += blocksum


def _k5(acco2, b2):
    return pl.pallas_call(
        _k5_body,
        grid=(8,),
        in_specs=[pl.BlockSpec((2, _BLK, 48), lambda i: (0, i, 0)),
                  pl.BlockSpec((1, 32), lambda i: (0, 0))],
        out_specs=pl.BlockSpec((1, 32), lambda i: (0, 0)),
        out_shape=jax.ShapeDtypeStruct((1, 32), _F32),
    )(acco2, b2)


def _head_kernel(pooled_ref, wr_ref, br_ref, o_ref):
    o_ref[...] = (jnp.sum(pooled_ref[...] * wr_ref[...][:, 0][None, :], axis=1,
                          keepdims=True) + br_ref[...][None, :])


def kernel(x, edge_index, W1, att_src1, att_dst1, b1, W2, att_src2, att_dst2,
           b2, Wr, br):
    n = x.shape[0]
    e = edge_index.shape[1]
    etot = e + n
    ep = -(-etot // (_NW * _CH)) * (_NW * _CH)
    cpt = ep // (_NW * _CH)

    loops = jnp.arange(n, dtype=edge_index.dtype)
    padi = jnp.full((ep - etot,), n, dtype=edge_index.dtype)
    srcx = jnp.concatenate([edge_index[0], loops, padi])
    dstx = jnp.concatenate([edge_index[1], loops, padi])

    x_pad = jnp.pad(x, ((0, _NP - n), (0, 0)))
    ar = jnp.arange(64)
    ar8 = jnp.arange(8)
    asw1 = jnp.zeros((64, 8), _F32).at[ar, ar // 8].set(att_src1.reshape(64))
    adw1 = jnp.zeros((64, 8), _F32).at[ar, ar // 8].set(att_dst1.reshape(64))
    rep = jnp.zeros((8, 64), _F32).at[ar // 8, ar].set(1.0)
    sel = jnp.zeros((64, 8), _F32).at[8 * ar8, ar8].set(1.0)
    z128 = jnp.zeros((_NP, 128), _F32)
    z48 = jnp.zeros((_NP, 48), _F32)

    t1, ad1r = _k1(x_pad, W1, asw1, adw1, rep)
    srcx2 = srcx.reshape(ep // _CH, _CH)
    dstx2 = dstx.reshape(ep // _CH, _CH)
    acco1 = _make_k2(cpt)(t1, ad1r, srcx2, dstx2, z128)
    t2, ad2r = _k3(acco1, b1.reshape(1, 64), W2, att_src2.reshape(32, 1),
                   att_dst2.reshape(32, 1), rep, sel)
    acco2 = _make_k4(cpt)(t2, ad2r, srcx2, dstx2, z48)
    pooled = _k5(acco2, b2.reshape(1, 32))
    out = pl.pallas_call(
        _head_kernel,
        out_shape=jax.ShapeDtypeStruct((1, 1), _F32),
    )(pooled, Wr, br)
    return out


# speedup vs baseline: 1.4775x; 1.0269x over previous
import functools

import jax
import jax.numpy as jnp
from jax import lax
from jax.experimental import pallas as pl
from jax.experimental.pallas import tpu as pltpu
from jax.experimental.pallas import tpu_sc as plsc

_F32 = jnp.float32
_I32 = jnp.int32

_N = 10000
_D = 128
_NP = 10112
_RPS = _NP // 16
_BLK = _NP // 8
_CH = 192
_NW = 32


def _dot(a, b, prec):
    return lax.dot_general(a, b, (((1,), (0,)), ((), ())), precision=prec,
                           preferred_element_type=_F32)


def _k1_body(x_ref, w_ref, asw_ref, adw_ref, rep_ref, t1_ref, ad1_ref):
    h = _dot(x_ref[...], w_ref[...], lax.Precision.DEFAULT)
    as1 = _dot(h, asw_ref[...], lax.Precision.HIGHEST)
    ad1 = _dot(h, adw_ref[...], lax.Precision.HIGHEST)
    as1r = _dot(as1, rep_ref[...], lax.Precision.HIGHEST)
    ad1r = _dot(ad1, rep_ref[...], lax.Precision.HIGHEST)
    t1_ref[...] = jnp.concatenate([h, as1r], axis=1)
    ad1_ref[...] = ad1r


def _k1(x_pad, W1, asw, adw, rep):
    return pl.pallas_call(
        _k1_body,
        grid=(8,),
        in_specs=[pl.BlockSpec((_BLK, _D), lambda i: (i, 0)),
                  pl.BlockSpec((_D, 64), lambda i: (0, 0)),
                  pl.BlockSpec((64, 8), lambda i: (0, 0)),
                  pl.BlockSpec((64, 8), lambda i: (0, 0)),
                  pl.BlockSpec((8, 64), lambda i: (0, 0))],
        out_specs=[pl.BlockSpec((_BLK, 128), lambda i: (i, 0)),
                   pl.BlockSpec((_BLK, 64), lambda i: (i, 0))],
        out_shape=[jax.ShapeDtypeStruct((_NP, 128), _F32),
                   jax.ShapeDtypeStruct((_NP, 64), _F32)],
    )(x_pad, W1, asw, adw, rep)


def _sc_mesh():
    return plsc.VectorSubcoreMesh(core_axis_name="c", subcore_axis_name="s")


def _make_k2(cpt):
    @functools.partial(
        pl.kernel,
        mesh=_sc_mesh(),
        compiler_params=pltpu.CompilerParams(use_tc_tiling_on_sc=False),
        out_type=jax.ShapeDtypeStruct((2, _NP, 128), _F32),
        scratch_types=[
            pltpu.VMEM((8, _CH), _I32),
            pltpu.VMEM((8, _CH), _I32),
            pltpu.VMEM((_CH, 128), _F32),
            pltpu.VMEM((_CH, 64), _F32),
            pltpu.VMEM_SHARED((_NP, 128), _F32),
            pltpu.SemaphoreType.DMA,
            pltpu.SemaphoreType.DMA,
        ],
    )
    def k2(t1, ad1, srcx2, dstx2, z128, acco, sidx_sl, didx_sl, rows_v, adr_v,
           acc_sh, sem1, sem2):
        c = lax.axis_index("c")
        s = lax.axis_index("s")
        wid = s * 2 + c
        ro = pl.multiple_of(s * _RPS, 8)
        pltpu.sync_copy(z128.at[pl.ds(ro, _RPS)], acc_sh.at[pl.ds(ro, _RPS)])
        plsc.subcore_barrier()

        def do_chunk(k):
            cp1 = pltpu.async_copy(t1.at[sidx_sl.at[k]], rows_v, sem1)
            cp2 = pltpu.async_copy(ad1.at[didx_sl.at[k]], adr_v, sem2)
            cp1.wait()
            cp2.wait()
            for i in range(_CH):
                for q in range(4):
                    lo = 16 * q
                    aq = rows_v[i, 64 + lo:80 + lo] + adr_v[i, lo:lo + 16]
                    aq = jnp.maximum(aq, 0.2 * aq)
                    mq = jnp.exp(aq)
                    rows_v[i, lo:lo + 16] = rows_v[i, lo:lo + 16] * mq
                    rows_v[i, 64 + lo:80 + lo] = mq
            pltpu.sync_copy(rows_v, acc_sh.at[didx_sl.at[k]], add=True)

        def load_slab(row, nrows):
            pltpu.sync_copy(srcx2.at[pl.ds(row, nrows)], sidx_sl.at[pl.ds(0, nrows)])
            pltpu.sync_copy(dstx2.at[pl.ds(row, nrows)], didx_sl.at[pl.ds(0, nrows)])

        def chunk_body(k, carry):
            do_chunk(k)
            return carry

        def slab(t, carry):
            load_slab(wid * cpt + t * 8, 8)
            lax.fori_loop(0, 8, chunk_body, 0)
            return carry

        lax.fori_loop(0, cpt // 8, slab, 0)
        tail = cpt % 8
        if tail:
            load_slab(wid * cpt + (cpt - tail), tail)
            lax.fori_loop(0, tail, chunk_body, 0)
        plsc.subcore_barrier()
        pltpu.sync_copy(acc_sh.at[pl.ds(ro, _RPS)], acco.at[c, pl.ds(ro, _RPS)])

    return k2


def _k3_body(acc_ref, b1_ref, w2_ref, asw_ref, adw_ref, rep_ref, sel_ref,
             t2_ref, ad2_ref):
    a = acc_ref[0] + acc_ref[1]
    den = jnp.maximum(_dot(a[:, 64:128], sel_ref[...],
                           lax.Precision.HIGHEST), 1e-30)
    denr = _dot(den, rep_ref[...], lax.Precision.HIGHEST)
    g = a[:, 0:64] / denr + b1_ref[...]
    g = jnp.where(g > 0, g, jnp.exp(g) - 1.0)
    h2 = _dot(g, w2_ref[...], lax.Precision.DEFAULT)
    as2 = _dot(h2, asw_ref[...], lax.Precision.HIGHEST)
    ad2 = _dot(h2, adw_ref[...], lax.Precision.HIGHEST)
    t2_ref[...] = jnp.concatenate(
        [h2, jnp.broadcast_to(as2, (as2.shape[0], 16))], axis=1)
    ad2_ref[...] = jnp.broadcast_to(ad2, (ad2.shape[0], 16))


def _k3(acco1, b1, W2, asw2, adw2, rep, sel):
    return pl.pallas_call(
        _k3_body,
        grid=(8,),
        in_specs=[pl.BlockSpec((2, _BLK, 128), lambda i: (0, i, 0)),
                  pl.BlockSpec((1, 64), lambda i: (0, 0)),
                  pl.BlockSpec((64, 32), lambda i: (0, 0)),
                  pl.BlockSpec((32, 1), lambda i: (0, 0)),
                  pl.BlockSpec((32, 1), lambda i: (0, 0)),
                  pl.BlockSpec((8, 64), lambda i: (0, 0)),
                  pl.BlockSpec((64, 8), lambda i: (0, 0))],
        out_specs=[pl.BlockSpec((_BLK, 48), lambda i: (i, 0)),
                   pl.BlockSpec((_BLK, 16), lambda i: (i, 0))],
        out_shape=[jax.ShapeDtypeStruct((_NP, 48), _F32),
                   jax.ShapeDtypeStruct((_NP, 16), _F32)],
    )(acco1, b1, W2, asw2, adw2, rep, sel)


def _make_k4(cpt):
    @functools.partial(
        pl.kernel,
        mesh=_sc_mesh(),
        compiler_params=pltpu.CompilerParams(use_tc_tiling_on_sc=False),
        out_type=jax.ShapeDtypeStruct((2, _NP, 48), _F32),
        scratch_types=[
            pltpu.VMEM((8, _CH), _I32),
            pltpu.VMEM((8, _CH), _I32),
            pltpu.VMEM((_CH, 48), _F32),
            pltpu.VMEM((_CH, 16), _F32),
            pltpu.VMEM_SHARED((_NP, 48), _F32),
            pltpu.SemaphoreType.DMA,
            pltpu.SemaphoreType.DMA,
        ],
    )
    def k4(t2, ad2, srcx2, dstx2, z48, acco, sidx_sl, didx_sl, rows_v, adr_v,
           acc_sh, sem1, sem2):
        c = lax.axis_index("c")
        s = lax.axis_index("s")
        wid = s * 2 + c
        ro = pl.multiple_of(s * _RPS, 8)
        pltpu.sync_copy(z48.at[pl.ds(ro, _RPS)], acc_sh.at[pl.ds(ro, _RPS)])
        plsc.subcore_barrier()

        def do_chunk(k):
            cp1 = pltpu.async_copy(t2.at[sidx_sl.at[k]], rows_v, sem1)
            cp2 = pltpu.async_copy(ad2.at[didx_sl.at[k]], adr_v, sem2)
            cp1.wait()
            cp2.wait()
            for i in range(_CH):
                a16 = rows_v[i, 32:48] + adr_v[i, 0:16]
                a16 = jnp.maximum(a16, 0.2 * a16)
                m = jnp.exp(a16)
                rows_v[i, 0:16] = rows_v[i, 0:16] * m
                rows_v[i, 16:32] = rows_v[i, 16:32] * m
                rows_v[i, 32:48] = m
            pltpu.sync_copy(rows_v, acc_sh.at[didx_sl.at[k]], add=True)

        def load_slab(row, nrows):
            pltpu.sync_copy(srcx2.at[pl.ds(row, nrows)], sidx_sl.at[pl.ds(0, nrows)])
            pltpu.sync_copy(dstx2.at[pl.ds(row, nrows)], didx_sl.at[pl.ds(0, nrows)])

        def chunk_body(k, carry):
            do_chunk(k)
            return carry

        def slab(t, carry):
            load_slab(wid * cpt + t * 8, 8)
            lax.fori_loop(0, 8, chunk_body, 0)
            return carry

        lax.fori_loop(0, cpt // 8, slab, 0)
        tail = cpt % 8
        if tail:
            load_slab(wid * cpt + (cpt - tail), tail)
            lax.fori_loop(0, tail, chunk_body, 0)
        plsc.subcore_barrier()
        pltpu.sync_copy(acc_sh.at[pl.ds(ro, _RPS)], acco.at[c, pl.ds(ro, _RPS)])

    return k4


def _k5_body(acc_ref, b2_ref, o_ref):
    i = pl.program_id(0)
    a = acc_ref[0] + acc_ref[1]
    den = jnp.maximum(a[:, 32:33], 1e-30)
    h2 = a[:, 0:32] / den + b2_ref[...]
    rowid = i * _BLK + lax.broadcasted_iota(_I32, (_BLK, 1), 0)
    h2 = jnp.where(rowid < _N, h2, 0.0)
    part = jnp.sum(h2.reshape(4, _BLK // 4, 32), axis=0)
    blocksum = jnp.sum(part, axis=0, keepdims=True)

    @pl.when(i == 0)
    def _():
        o_ref[...] = blocksum

    @pl.when(i > 0)
    def _():
        o_ref[...] += blocksum


def _k5(acco2, b2):
    return pl.pallas_call(
        _k5_body,
        grid=(8,),
        in_specs=[pl.BlockSpec((2, _BLK, 48), lambda i: (0, i, 0)),
                  pl.BlockSpec((1, 32), lambda i: (0, 0))],
        out_specs=pl.BlockSpec((1, 32), lambda i: (0, 0)),
        out_shape=jax.ShapeDtypeStruct((1, 32), _F32),
    )(acco2, b2)


def _head_kernel(pooled_ref, wr_ref, br_ref, o_ref):
    o_ref[...] = (jnp.sum(pooled_ref[...] * wr_ref[...][:, 0][None, :], axis=1,
                          keepdims=True) + br_ref[...][None, :])


def kernel(x, edge_index, W1, att_src1, att_dst1, b1, W2, att_src2, att_dst2,
           b2, Wr, br):
    n = x.shape[0]
    e = edge_index.shape[1]
    etot = e + n
    ep = -(-etot // (_NW * _CH)) * (_NW * _CH)
    cpt = ep // (_NW * _CH)

    loops = jnp.arange(n, dtype=edge_index.dtype)
    padi = jnp.full((ep - etot,), n, dtype=edge_index.dtype)
    srcx = jnp.concatenate([edge_index[0], loops, padi])
    dstx = jnp.concatenate([edge_index[1], loops, padi])

    x_pad = jnp.pad(x, ((0, _NP - n), (0, 0)))
    ar = jnp.arange(64)
    ar8 = jnp.arange(8)
    asw1 = jnp.zeros((64, 8), _F32).at[ar, ar // 8].set(att_src1.reshape(64))
    adw1 = jnp.zeros((64, 8), _F32).at[ar, ar // 8].set(att_dst1.reshape(64))
    rep = jnp.zeros((8, 64), _F32).at[ar // 8, ar].set(1.0)
    sel = jnp.zeros((64, 8), _F32).at[8 * ar8, ar8].set(1.0)
    z128 = jnp.zeros((_NP, 128), _F32)
    z48 = jnp.zeros((_NP, 48), _F32)

    t1, ad1r = _k1(x_pad, W1, asw1, adw1, rep)
    srcx2 = srcx.reshape(ep // _CH, _CH)
    dstx2 = dstx.reshape(ep // _CH, _CH)
    acco1 = _make_k2(cpt)(t1, ad1r, srcx2, dstx2, z128)
    t2, ad2r = _k3(acco1, b1.reshape(1, 64), W2, att_src2.reshape(32, 1),
                   att_dst2.reshape(32, 1), rep, sel)
    acco2 = _make_k4(cpt)(t2, ad2r, srcx2, dstx2, z48)
    pooled = _k5(acco2, b2.reshape(1, 32))
    out = pl.pallas_call(
        _head_kernel,
        out_shape=jax.ShapeDtypeStruct((1, 1), _F32),
    )(pooled, Wr, br)
    return out


# speedup vs baseline: 1.4891x; 1.0078x over previous
import functools

import jax
import jax.numpy as jnp
from jax import lax
from jax.experimental import pallas as pl
from jax.experimental.pallas import tpu as pltpu
from jax.experimental.pallas import tpu_sc as plsc

_F32 = jnp.float32
_I32 = jnp.int32

_N = 10000
_D = 128
_NP = 10112
_RPS = _NP // 16
_BLK = _NP // 8
_CH = 192
_CH4 = 384
_NW = 32


def _dot(a, b, prec):
    return lax.dot_general(a, b, (((1,), (0,)), ((), ())), precision=prec,
                           preferred_element_type=_F32)


def _k1_body(x_ref, w_ref, asw_ref, adw_ref, rep_ref, t1_ref, ad1_ref):
    h = _dot(x_ref[...], w_ref[...], lax.Precision.DEFAULT)
    as1 = _dot(h, asw_ref[...], lax.Precision.HIGHEST)
    ad1 = _dot(h, adw_ref[...], lax.Precision.HIGHEST)
    as1r = _dot(as1, rep_ref[...], lax.Precision.HIGHEST)
    ad1r = _dot(ad1, rep_ref[...], lax.Precision.HIGHEST)
    t1_ref[...] = jnp.concatenate([h, as1r], axis=1)
    ad1_ref[...] = ad1r


def _k1(x_pad, W1, asw, adw, rep):
    return pl.pallas_call(
        _k1_body,
        grid=(8,),
        in_specs=[pl.BlockSpec((_BLK, _D), lambda i: (i, 0)),
                  pl.BlockSpec((_D, 64), lambda i: (0, 0)),
                  pl.BlockSpec((64, 8), lambda i: (0, 0)),
                  pl.BlockSpec((64, 8), lambda i: (0, 0)),
                  pl.BlockSpec((8, 64), lambda i: (0, 0))],
        out_specs=[pl.BlockSpec((_BLK, 128), lambda i: (i, 0)),
                   pl.BlockSpec((_BLK, 64), lambda i: (i, 0))],
        out_shape=[jax.ShapeDtypeStruct((_NP, 128), _F32),
                   jax.ShapeDtypeStruct((_NP, 64), _F32)],
    )(x_pad, W1, asw, adw, rep)


def _sc_mesh():
    return plsc.VectorSubcoreMesh(core_axis_name="c", subcore_axis_name="s")


def _make_k2(cpt):
    @functools.partial(
        pl.kernel,
        mesh=_sc_mesh(),
        compiler_params=pltpu.CompilerParams(use_tc_tiling_on_sc=False),
        out_type=jax.ShapeDtypeStruct((2, _NP, 128), _F32),
        scratch_types=[
            pltpu.VMEM((8, _CH), _I32),
            pltpu.VMEM((8, _CH), _I32),
            pltpu.VMEM((_CH, 128), _F32),
            pltpu.VMEM((_CH, 64), _F32),
            pltpu.VMEM_SHARED((_NP, 128), _F32),
            pltpu.SemaphoreType.DMA,
            pltpu.SemaphoreType.DMA,
        ],
    )
    def k2(t1, ad1, srcx2, dstx2, z128, acco, sidx_sl, didx_sl, rows_v, adr_v,
           acc_sh, sem1, sem2):
        c = lax.axis_index("c")
        s = lax.axis_index("s")
        wid = s * 2 + c
        ro = pl.multiple_of(s * _RPS, 8)
        pltpu.sync_copy(z128.at[pl.ds(ro, _RPS)], acc_sh.at[pl.ds(ro, _RPS)])
        plsc.subcore_barrier()

        def do_chunk(k):
            cp1 = pltpu.async_copy(t1.at[sidx_sl.at[k]], rows_v, sem1)
            cp2 = pltpu.async_copy(ad1.at[didx_sl.at[k]], adr_v, sem2)
            cp1.wait()
            cp2.wait()
            for i in range(_CH):
                for q in range(4):
                    lo = 16 * q
                    aq = rows_v[i, 64 + lo:80 + lo] + adr_v[i, lo:lo + 16]
                    aq = jnp.maximum(aq, 0.2 * aq)
                    mq = jnp.exp(aq)
                    rows_v[i, lo:lo + 16] = rows_v[i, lo:lo + 16] * mq
                    rows_v[i, 64 + lo:80 + lo] = mq
            pltpu.sync_copy(rows_v, acc_sh.at[didx_sl.at[k]], add=True)

        def load_slab(row, nrows):
            pltpu.sync_copy(srcx2.at[pl.ds(row, nrows)], sidx_sl.at[pl.ds(0, nrows)])
            pltpu.sync_copy(dstx2.at[pl.ds(row, nrows)], didx_sl.at[pl.ds(0, nrows)])

        def chunk_body(k, carry):
            do_chunk(k)
            return carry

        def slab(t, carry):
            load_slab(wid * cpt + t * 8, 8)
            lax.fori_loop(0, 8, chunk_body, 0)
            return carry

        lax.fori_loop(0, cpt // 8, slab, 0)
        tail = cpt % 8
        if tail:
            load_slab(wid * cpt + (cpt - tail), tail)
            lax.fori_loop(0, tail, chunk_body, 0)
        plsc.subcore_barrier()
        pltpu.sync_copy(acc_sh.at[pl.ds(ro, _RPS)], acco.at[c, pl.ds(ro, _RPS)])

    return k2


def _k3_body(acc_ref, b1_ref, w2_ref, asw_ref, adw_ref, rep_ref, sel_ref,
             t2_ref, ad2_ref):
    a = acc_ref[0] + acc_ref[1]
    den = jnp.maximum(_dot(a[:, 64:128], sel_ref[...],
                           lax.Precision.HIGHEST), 1e-30)
    denr = _dot(den, rep_ref[...], lax.Precision.HIGHEST)
    g = a[:, 0:64] / denr + b1_ref[...]
    g = jnp.where(g > 0, g, jnp.exp(g) - 1.0)
    h2 = _dot(g, w2_ref[...], lax.Precision.DEFAULT)
    as2 = _dot(h2, asw_ref[...], lax.Precision.HIGHEST)
    ad2 = _dot(h2, adw_ref[...], lax.Precision.HIGHEST)
    t2_ref[...] = jnp.concatenate(
        [h2, jnp.broadcast_to(as2, (as2.shape[0], 16))], axis=1)
    ad2_ref[...] = jnp.broadcast_to(ad2, (ad2.shape[0], 16))


def _k3(acco1, b1, W2, asw2, adw2, rep, sel):
    return pl.pallas_call(
        _k3_body,
        grid=(8,),
        in_specs=[pl.BlockSpec((2, _BLK, 128), lambda i: (0, i, 0)),
                  pl.BlockSpec((1, 64), lambda i: (0, 0)),
                  pl.BlockSpec((64, 32), lambda i: (0, 0)),
                  pl.BlockSpec((32, 1), lambda i: (0, 0)),
                  pl.BlockSpec((32, 1), lambda i: (0, 0)),
                  pl.BlockSpec((8, 64), lambda i: (0, 0)),
                  pl.BlockSpec((64, 8), lambda i: (0, 0))],
        out_specs=[pl.BlockSpec((_BLK, 48), lambda i: (i, 0)),
                   pl.BlockSpec((_BLK, 16), lambda i: (i, 0))],
        out_shape=[jax.ShapeDtypeStruct((_NP, 48), _F32),
                   jax.ShapeDtypeStruct((_NP, 16), _F32)],
    )(acco1, b1, W2, asw2, adw2, rep, sel)


def _make_k4(cpt):
    @functools.partial(
        pl.kernel,
        mesh=_sc_mesh(),
        compiler_params=pltpu.CompilerParams(use_tc_tiling_on_sc=False),
        out_type=jax.ShapeDtypeStruct((2, _NP, 48), _F32),
        scratch_types=[
            pltpu.VMEM((8, _CH4), _I32),
            pltpu.VMEM((8, _CH4), _I32),
            pltpu.VMEM((_CH4, 48), _F32),
            pltpu.VMEM((_CH4, 16), _F32),
            pltpu.VMEM_SHARED((_NP, 48), _F32),
            pltpu.SemaphoreType.DMA,
            pltpu.SemaphoreType.DMA,
        ],
    )
    def k4(t2, ad2, srcx2, dstx2, z48, acco, sidx_sl, didx_sl, rows_v, adr_v,
           acc_sh, sem1, sem2):
        c = lax.axis_index("c")
        s = lax.axis_index("s")
        wid = s * 2 + c
        ro = pl.multiple_of(s * _RPS, 8)
        pltpu.sync_copy(z48.at[pl.ds(ro, _RPS)], acc_sh.at[pl.ds(ro, _RPS)])
        plsc.subcore_barrier()

        def do_chunk(k):
            cp1 = pltpu.async_copy(t2.at[sidx_sl.at[k]], rows_v, sem1)
            cp2 = pltpu.async_copy(ad2.at[didx_sl.at[k]], adr_v, sem2)
            cp1.wait()
            cp2.wait()
            for i in range(_CH4):
                a16 = rows_v[i, 32:48] + adr_v[i, 0:16]
                a16 = jnp.maximum(a16, 0.2 * a16)
                m = jnp.exp(a16)
                rows_v[i, 0:16] = rows_v[i, 0:16] * m
                rows_v[i, 16:32] = rows_v[i, 16:32] * m
                rows_v[i, 32:48] = m
            pltpu.sync_copy(rows_v, acc_sh.at[didx_sl.at[k]], add=True)

        def load_slab(row, nrows):
            pltpu.sync_copy(srcx2.at[pl.ds(row, nrows)], sidx_sl.at[pl.ds(0, nrows)])
            pltpu.sync_copy(dstx2.at[pl.ds(row, nrows)], didx_sl.at[pl.ds(0, nrows)])

        def chunk_body(k, carry):
            do_chunk(k)
            return carry

        def slab(t, carry):
            load_slab(wid * cpt + t * 8, 8)
            lax.fori_loop(0, 8, chunk_body, 0)
            return carry

        lax.fori_loop(0, cpt // 8, slab, 0)
        tail = cpt % 8
        if tail:
            load_slab(wid * cpt + (cpt - tail), tail)
            lax.fori_loop(0, tail, chunk_body, 0)
        plsc.subcore_barrier()
        pltpu.sync_copy(acc_sh.at[pl.ds(ro, _RPS)], acco.at[c, pl.ds(ro, _RPS)])

    return k4


def _k5_body(acc_ref, b2_ref, wr_ref, br_ref, o_ref, pool_ref):
    i = pl.program_id(0)
    a = acc_ref[0] + acc_ref[1]
    den = jnp.maximum(a[:, 32:33], 1e-30)
    h2 = a[:, 0:32] / den + b2_ref[...]
    rowid = i * _BLK + lax.broadcasted_iota(_I32, (_BLK, 1), 0)
    h2 = jnp.where(rowid < _N, h2, 0.0)
    part = jnp.sum(h2.reshape(4, _BLK // 4, 32), axis=0)
    blocksum = jnp.sum(part, axis=0, keepdims=True)

    @pl.when(i == 0)
    def _():
        pool_ref[...] = blocksum

    @pl.when(i > 0)
    def _():
        pool_ref[...] += blocksum

    @pl.when(i == 7)
    def _():
        o_ref[...] = (jnp.sum(pool_ref[...] * wr_ref[...][:, 0][None, :],
                              axis=1, keepdims=True) + br_ref[...][None, :])


def _k5(acco2, b2, Wr, br):
    return pl.pallas_call(
        _k5_body,
        grid=(8,),
        in_specs=[pl.BlockSpec((2, _BLK, 48), lambda i: (0, i, 0)),
                  pl.BlockSpec((1, 32), lambda i: (0, 0)),
                  pl.BlockSpec((32, 1), lambda i: (0, 0)),
                  pl.BlockSpec((1,), lambda i: (0,))],
        out_specs=pl.BlockSpec((1, 1), lambda i: (0, 0)),
        out_shape=jax.ShapeDtypeStruct((1, 1), _F32),
        scratch_shapes=[pltpu.VMEM((1, 32), _F32)],
    )(acco2, b2, Wr, br)


def _head_kernel(pooled_ref, wr_ref, br_ref, o_ref):
    o_ref[...] = (jnp.sum(pooled_ref[...] * wr_ref[...][:, 0][None, :], axis=1,
                          keepdims=True) + br_ref[...][None, :])


def kernel(x, edge_index, W1, att_src1, att_dst1, b1, W2, att_src2, att_dst2,
           b2, Wr, br):
    n = x.shape[0]
    e = edge_index.shape[1]
    etot = e + n
    ep = -(-etot // (_NW * _CH4)) * (_NW * _CH4)
    cpt = ep // (_NW * _CH)
    cpt4 = ep // (_NW * _CH4)

    loops = jnp.arange(n, dtype=edge_index.dtype)
    padi = jnp.full((ep - etot,), n, dtype=edge_index.dtype)
    srcx = jnp.concatenate([edge_index[0], loops, padi])
    dstx = jnp.concatenate([edge_index[1], loops, padi])

    x_pad = jnp.pad(x, ((0, _NP - n), (0, 0)))
    ar = jnp.arange(64)
    ar8 = jnp.arange(8)
    asw1 = jnp.zeros((64, 8), _F32).at[ar, ar // 8].set(att_src1.reshape(64))
    adw1 = jnp.zeros((64, 8), _F32).at[ar, ar // 8].set(att_dst1.reshape(64))
    rep = jnp.zeros((8, 64), _F32).at[ar // 8, ar].set(1.0)
    sel = jnp.zeros((64, 8), _F32).at[8 * ar8, ar8].set(1.0)
    z128 = jnp.zeros((_NP, 128), _F32)
    z48 = jnp.zeros((_NP, 48), _F32)

    t1, ad1r = _k1(x_pad, W1, asw1, adw1, rep)
    srcx2 = srcx.reshape(ep // _CH, _CH)
    dstx2 = dstx.reshape(ep // _CH, _CH)
    acco1 = _make_k2(cpt)(t1, ad1r, srcx2, dstx2, z128)
    t2, ad2r = _k3(acco1, b1.reshape(1, 64), W2, att_src2.reshape(32, 1),
                   att_dst2.reshape(32, 1), rep, sel)
    srcx4 = srcx.reshape(ep // _CH4, _CH4)
    dstx4 = dstx.reshape(ep // _CH4, _CH4)
    acco2 = _make_k4(cpt4)(t2, ad2r, srcx4, dstx4, z48)
    return _k5(acco2, b2.reshape(1, 32), Wr, br)
